# baseline (speedup 1.0000x reference)
import jax
import jax.numpy as jnp
from jax import lax
from jax.experimental import pallas as pl
from jax.experimental.pallas import tpu as pltpu
from jax.experimental.pallas import tpu_sc as plsc

N_NODES = 10000
N_EDGES = 320000
DIM = 128
NUM_CLASSES = 40

NC = 2
NS = 16
DH = DIM // NC
RPT = N_NODES // NS
EC = 40
ER = N_EDGES // EC
ERT = ER // NS
DEC = 80
DER = N_EDGES // DEC
ERW = DER // (NC * NS)
DW = 16

RB = 1000
GRID = N_NODES // RB


def _sc_mesh():
  return plsc.VectorSubcoreMesh(
      core_axis_name="c", subcore_axis_name="s",
      num_cores=NC, num_subcores=NS)


def _deg_body(ei3, degp, acc, dst2d, ones, zbuf):
  c = lax.axis_index("c")
  s = lax.axis_index("s")
  w = c * NS + s
  r0 = s * RPT

  def fill_ones(i, carry):
    ones[i, :] = jnp.full((DW,), 1.0, dtype=jnp.float32)
    return carry
  lax.fori_loop(0, DEC, fill_ones, 0)

  def fill_zero(i, carry):
    zbuf[i, :] = jnp.zeros((DW,), dtype=jnp.float32)
    return carry
  lax.fori_loop(0, RPT, fill_zero, 0)

  pltpu.sync_copy(zbuf, acc.at[pl.ds(r0, RPT)])
  pltpu.sync_copy(ei3.at[1, pl.ds(w * ERW, ERW)], dst2d)
  plsc.subcore_barrier()

  def chunk(i, carry):
    pltpu.sync_copy(ones, acc.at[dst2d.at[i]], add=True)
    return carry
  lax.fori_loop(0, ERW, chunk, 0)

  plsc.subcore_barrier()
  pltpu.sync_copy(acc.at[pl.ds(r0, RPT)], degp.at[c, pl.ds(r0, RPT)])


def _degrees(ei3):
  f = pl.kernel(
      _deg_body,
      out_type=jax.ShapeDtypeStruct((NC, N_NODES, DW), jnp.float32),
      mesh=_sc_mesh(),
      scratch_types=[
          pltpu.VMEM_SHARED((N_NODES, DW), jnp.float32),
          pltpu.VMEM((ERW, DEC), jnp.int32),
          pltpu.VMEM((DEC, DW), jnp.float32),
          pltpu.VMEM((RPT, DW), jnp.float32),
      ],
      compiler_params=pltpu.CompilerParams(use_tc_tiling_on_sc=False),
      name="gcn_degrees",
  )
  return f(ei3)


def _prop_body(hsa, hsb, ei3, out, table, acc, src2d, dst2d,
               rows0, rows1,
               gsem0, gsem1,
               ssem0, ssem1):
  c = lax.axis_index("c")
  s = lax.axis_index("s")
  r0 = s * RPT
  col0 = c * DH
  rows = (rows0, rows1)
  gsem = (gsem0, gsem1)
  ssem = (ssem0, ssem1)

  def stage(hs):
    pltpu.async_copy(hs.at[pl.ds(r0, RPT)], table.at[pl.ds(r0, RPT)], gsem[0])
    pltpu.async_copy(hs.at[pl.ds(r0, RPT)], acc.at[pl.ds(r0, RPT)], gsem[1])
    pltpu.async_copy(ei3.at[0, pl.ds(s * ERT, ERT)], src2d, ssem[0])
    pltpu.async_copy(ei3.at[1, pl.ds(s * ERT, ERT)], dst2d, ssem[1])
    pltpu.make_async_copy(hs.at[pl.ds(r0, RPT)], table.at[pl.ds(r0, RPT)],
                          gsem[0]).wait()
    pltpu.make_async_copy(hs.at[pl.ds(r0, RPT)], acc.at[pl.ds(r0, RPT)],
                          gsem[1]).wait()
    pltpu.make_async_copy(ei3.at[0, pl.ds(s * ERT, ERT)], src2d,
                          ssem[0]).wait()
    pltpu.make_async_copy(ei3.at[1, pl.ds(s * ERT, ERT)], dst2d,
                          ssem[1]).wait()

  @pl.when(c == 0)
  def _():
    stage(hsa)

  @pl.when(c == 1)
  def _():
    stage(hsb)

  plsc.subcore_barrier()

  def gather_start(i, b):
    pltpu.async_copy(table.at[src2d.at[i]], rows[b], gsem[b])

  def gather_wait(i, b):
    pltpu.make_async_copy(table.at[src2d.at[i]], rows[b], gsem[b]).wait()

  gather_start(0, 0)

  def chunk(it, carry):
    g = it * 2
    gather_wait(g, 0)
    gather_start(g + 1, 1)
    pltpu.sync_copy(rows[0], acc.at[dst2d.at[g]], add=True)
    gather_wait(g + 1, 1)

    @pl.when(g + 2 < ERT)
    def _():
      gather_start(g + 2, 0)

    pltpu.sync_copy(rows[1], acc.at[dst2d.at[g + 1]], add=True)
    return carry
  lax.fori_loop(0, ERT // 2, chunk, 0, unroll=False)

  plsc.subcore_barrier()
  pltpu.sync_copy(acc.at[pl.ds(r0, RPT)],
                  out.at[pl.ds(r0, RPT), pl.ds(col0, DH)])


def _propagate(hsa, hsb, ei3):
  f = pl.kernel(
      _prop_body,
      out_type=jax.ShapeDtypeStruct((N_NODES, DIM), jnp.float32),
      mesh=_sc_mesh(),
      scratch_types=[
          pltpu.VMEM_SHARED((N_NODES, DH), jnp.float32),
          pltpu.VMEM_SHARED((N_NODES, DH), jnp.float32),
          pltpu.VMEM((ERT, EC), jnp.int32),
          pltpu.VMEM((ERT, EC), jnp.int32),
          pltpu.VMEM((EC, DH), jnp.float32),
          pltpu.VMEM((EC, DH), jnp.float32),
          pltpu.SemaphoreType.DMA,
          pltpu.SemaphoreType.DMA,
          pltpu.SemaphoreType.DMA,
          pltpu.SemaphoreType.DMA,
      ],
      compiler_params=pltpu.CompilerParams(use_tc_tiling_on_sc=False),
      name="gcn_propagate",
  )
  return f(hsa, hsb, ei3)


def _dinv_of(degp_ref):
  deg = degp_ref[0, :, 0] + degp_ref[1, :, 0] + 1.0
  return lax.rsqrt(deg)


def _mm1_body(x_ref, w1_ref, o_ref):
  o_ref[...] = jnp.dot(x_ref[...], w1_ref[...],
                       preferred_element_type=jnp.float32,
                       precision=lax.Precision.HIGHEST)


def _tc1_body(degp_ref, h_ref, oa_ref, ob_ref):
  dinv = _dinv_of(degp_ref)
  h = h_ref[...] * dinv[:, None]
  oa_ref[...] = h[:, :DH]
  ob_ref[...] = h[:, DH:]


def _tc2_body(degp_ref, s1_ref, b1_ref, w2_ref, oa_ref, ob_ref):
  dinv = _dinv_of(degp_ref)
  h1p = jnp.maximum(s1_ref[...] * dinv[:, None] + b1_ref[...], 0.0)
  h2 = jnp.dot(h1p, w2_ref[...],
               preferred_element_type=jnp.float32,
               precision=lax.Precision.HIGHEST) * dinv[:, None]
  oa_ref[...] = h2[:, :DH]
  ob_ref[...] = h2[:, DH:]


def _tc3_body(degp_ref, s2_ref, b2_ref, wl_ref, bl_ref, o_ref):
  dinv = _dinv_of(degp_ref)
  h2p = jnp.maximum(s2_ref[...] * dinv[:, None] + b2_ref[...], 0.0)
  o_ref[...] = jnp.dot(h2p, wl_ref[...],
                       preferred_element_type=jnp.float32,
                       precision=lax.Precision.HIGHEST) + bl_ref[...]


def _degp_spec():
  return pl.BlockSpec((NC, RB, DW), lambda i: (0, i, 0))


def _mm1(x, W1):
  return pl.pallas_call(
      _mm1_body,
      grid=(GRID,),
      in_specs=[
          pl.BlockSpec((RB, DIM), lambda i: (i, 0)),
          pl.BlockSpec((DIM, DIM), lambda i: (0, 0)),
      ],
      out_specs=pl.BlockSpec((RB, DIM), lambda i: (i, 0)),
      out_shape=jax.ShapeDtypeStruct((N_NODES, DIM), jnp.float32),
  )(x, W1)


def _tc_stage1(degp, h1):
  return pl.pallas_call(
      _tc1_body,
      grid=(GRID,),
      in_specs=[
          _degp_spec(),
          pl.BlockSpec((RB, DIM), lambda i: (i, 0)),
      ],
      out_specs=[pl.BlockSpec((RB, DH), lambda i: (i, 0)),
                 pl.BlockSpec((RB, DH), lambda i: (i, 0))],
      out_shape=[jax.ShapeDtypeStruct((N_NODES, DH), jnp.float32),
                 jax.ShapeDtypeStruct((N_NODES, DH), jnp.float32)],
  )(degp, h1)


def _tc_stage2(degp, s1, b1, W2):
  return pl.pallas_call(
      _tc2_body,
      grid=(GRID,),
      in_specs=[
          _degp_spec(),
          pl.BlockSpec((RB, DIM), lambda i: (i, 0)),
          pl.BlockSpec((1, DIM), lambda i: (0, 0)),
          pl.BlockSpec((DIM, DIM), lambda i: (0, 0)),
      ],
      out_specs=[pl.BlockSpec((RB, DH), lambda i: (i, 0)),
                 pl.BlockSpec((RB, DH), lambda i: (i, 0))],
      out_shape=[jax.ShapeDtypeStruct((N_NODES, DH), jnp.float32),
                 jax.ShapeDtypeStruct((N_NODES, DH), jnp.float32)],
  )(degp, s1, b1, W2)


def _tc_stage3(degp, s2, b2, Wl, bl):
  return pl.pallas_call(
      _tc3_body,
      grid=(GRID,),
      in_specs=[
          _degp_spec(),
          pl.BlockSpec((RB, DIM), lambda i: (i, 0)),
          pl.BlockSpec((1, DIM), lambda i: (0, 0)),
          pl.BlockSpec((DIM, NUM_CLASSES), lambda i: (0, 0)),
          pl.BlockSpec((1, NUM_CLASSES), lambda i: (0, 0)),
      ],
      out_specs=pl.BlockSpec((RB, NUM_CLASSES), lambda i: (i, 0)),
      out_shape=jax.ShapeDtypeStruct((N_NODES, NUM_CLASSES), jnp.float32),
  )(degp, s2, b2, Wl, bl)


def kernel(x, edge_index, W1, b1, W2, b2, Wl, bl):
  ei3 = edge_index.reshape(2, ER, EC)
  h1 = _mm1(x, W1)
  degp = _degrees(edge_index.reshape(2, DER, DEC))
  hs1a, hs1b = _tc_stage1(degp, h1)
  s1 = _propagate(hs1a, hs1b, ei3)
  hs2a, hs2b = _tc_stage2(degp, s1, b1.reshape(1, DIM), W2)
  s2 = _propagate(hs2a, hs2b, ei3)
  return _tc_stage3(degp, s2, b2.reshape(1, DIM), Wl, bl.reshape(1, NUM_CLASSES))


# speedup vs baseline: 1.0210x; 1.0210x over previous
import jax
import jax.numpy as jnp
from jax import lax
from jax.experimental import pallas as pl
from jax.experimental.pallas import tpu as pltpu
from jax.experimental.pallas import tpu_sc as plsc

N_NODES = 10000
N_EDGES = 320000
DIM = 128
NUM_CLASSES = 40

NC = 2
NS = 16
DH = DIM // NC
RPT = N_NODES // NS
EC = 40
ER = N_EDGES // EC
ERT = ER // NS
DEC = 80
DER = N_EDGES // DEC
ERW = DER // (NC * NS)
DW = 16

RB = 10000
GRID = N_NODES // RB


def _sc_mesh():
  return plsc.VectorSubcoreMesh(
      core_axis_name="c", subcore_axis_name="s",
      num_cores=NC, num_subcores=NS)


def _deg_body(ei3, degp, acc, dst2d, ones, zbuf):
  c = lax.axis_index("c")
  s = lax.axis_index("s")
  w = c * NS + s
  r0 = s * RPT

  def fill_ones(i, carry):
    ones[i, :] = jnp.full((DW,), 1.0, dtype=jnp.float32)
    return carry
  lax.fori_loop(0, DEC, fill_ones, 0)

  def fill_zero(i, carry):
    zbuf[i, :] = jnp.zeros((DW,), dtype=jnp.float32)
    return carry
  lax.fori_loop(0, RPT, fill_zero, 0)

  pltpu.sync_copy(zbuf, acc.at[pl.ds(r0, RPT)])
  pltpu.sync_copy(ei3.at[1, pl.ds(w * ERW, ERW)], dst2d)
  plsc.subcore_barrier()

  def chunk(i, carry):
    pltpu.sync_copy(ones, acc.at[dst2d.at[i]], add=True)
    return carry
  lax.fori_loop(0, ERW, chunk, 0)

  plsc.subcore_barrier()
  pltpu.sync_copy(acc.at[pl.ds(r0, RPT)], degp.at[c, pl.ds(r0, RPT)])


def _degrees(ei3):
  f = pl.kernel(
      _deg_body,
      out_type=jax.ShapeDtypeStruct((NC, N_NODES, DW), jnp.float32),
      mesh=_sc_mesh(),
      scratch_types=[
          pltpu.VMEM_SHARED((N_NODES, DW), jnp.float32),
          pltpu.VMEM((ERW, DEC), jnp.int32),
          pltpu.VMEM((DEC, DW), jnp.float32),
          pltpu.VMEM((RPT, DW), jnp.float32),
      ],
      compiler_params=pltpu.CompilerParams(use_tc_tiling_on_sc=False),
      name="gcn_degrees",
  )
  return f(ei3)


def _prop_body(hsa, hsb, ei3, out, table, acc, src2d, dst2d,
               rows0, rows1,
               gsem0, gsem1,
               ssem0, ssem1):
  c = lax.axis_index("c")
  s = lax.axis_index("s")
  r0 = s * RPT
  col0 = c * DH
  rows = (rows0, rows1)
  gsem = (gsem0, gsem1)
  ssem = (ssem0, ssem1)

  def stage(hs):
    pltpu.async_copy(hs.at[pl.ds(r0, RPT)], table.at[pl.ds(r0, RPT)], gsem[0])
    pltpu.async_copy(hs.at[pl.ds(r0, RPT)], acc.at[pl.ds(r0, RPT)], gsem[1])
    pltpu.async_copy(ei3.at[0, pl.ds(s * ERT, ERT)], src2d, ssem[0])
    pltpu.async_copy(ei3.at[1, pl.ds(s * ERT, ERT)], dst2d, ssem[1])
    pltpu.make_async_copy(hs.at[pl.ds(r0, RPT)], table.at[pl.ds(r0, RPT)],
                          gsem[0]).wait()
    pltpu.make_async_copy(hs.at[pl.ds(r0, RPT)], acc.at[pl.ds(r0, RPT)],
                          gsem[1]).wait()
    pltpu.make_async_copy(ei3.at[0, pl.ds(s * ERT, ERT)], src2d,
                          ssem[0]).wait()
    pltpu.make_async_copy(ei3.at[1, pl.ds(s * ERT, ERT)], dst2d,
                          ssem[1]).wait()

  @pl.when(c == 0)
  def _():
    stage(hsa)

  @pl.when(c == 1)
  def _():
    stage(hsb)

  plsc.subcore_barrier()

  def gather_start(i, b):
    pltpu.async_copy(table.at[src2d.at[i]], rows[b], gsem[b])

  def gather_wait(i, b):
    pltpu.make_async_copy(table.at[src2d.at[i]], rows[b], gsem[b]).wait()

  gather_start(0, 0)

  def chunk(it, carry):
    g = it * 2
    gather_wait(g, 0)
    gather_start(g + 1, 1)
    pltpu.sync_copy(rows[0], acc.at[dst2d.at[g]], add=True)
    gather_wait(g + 1, 1)

    @pl.when(g + 2 < ERT)
    def _():
      gather_start(g + 2, 0)

    pltpu.sync_copy(rows[1], acc.at[dst2d.at[g + 1]], add=True)
    return carry
  lax.fori_loop(0, ERT // 2, chunk, 0, unroll=False)

  plsc.subcore_barrier()
  pltpu.sync_copy(acc.at[pl.ds(r0, RPT)],
                  out.at[pl.ds(r0, RPT), pl.ds(col0, DH)])


def _propagate(hsa, hsb, ei3):
  f = pl.kernel(
      _prop_body,
      out_type=jax.ShapeDtypeStruct((N_NODES, DIM), jnp.float32),
      mesh=_sc_mesh(),
      scratch_types=[
          pltpu.VMEM_SHARED((N_NODES, DH), jnp.float32),
          pltpu.VMEM_SHARED((N_NODES, DH), jnp.float32),
          pltpu.VMEM((ERT, EC), jnp.int32),
          pltpu.VMEM((ERT, EC), jnp.int32),
          pltpu.VMEM((EC, DH), jnp.float32),
          pltpu.VMEM((EC, DH), jnp.float32),
          pltpu.SemaphoreType.DMA,
          pltpu.SemaphoreType.DMA,
          pltpu.SemaphoreType.DMA,
          pltpu.SemaphoreType.DMA,
      ],
      compiler_params=pltpu.CompilerParams(use_tc_tiling_on_sc=False),
      name="gcn_propagate",
  )
  return f(hsa, hsb, ei3)


def _dinv_of(degp_ref):
  deg = degp_ref[0, :, 0] + degp_ref[1, :, 0] + 1.0
  return lax.rsqrt(deg)


def _mm1_body(x_ref, w1_ref, o_ref):
  o_ref[...] = jnp.dot(x_ref[...], w1_ref[...],
                       preferred_element_type=jnp.float32,
                       precision=lax.Precision.HIGHEST)


def _tc1_body(degp_ref, h_ref, oa_ref, ob_ref):
  dinv = _dinv_of(degp_ref)
  h = h_ref[...] * dinv[:, None]
  oa_ref[...] = h[:, :DH]
  ob_ref[...] = h[:, DH:]


def _tc2_body(degp_ref, s1_ref, b1_ref, w2_ref, oa_ref, ob_ref):
  dinv = _dinv_of(degp_ref)
  h1p = jnp.maximum(s1_ref[...] * dinv[:, None] + b1_ref[...], 0.0)
  h2 = jnp.dot(h1p, w2_ref[...],
               preferred_element_type=jnp.float32,
               precision=lax.Precision.HIGHEST) * dinv[:, None]
  oa_ref[...] = h2[:, :DH]
  ob_ref[...] = h2[:, DH:]


def _tc3_body(degp_ref, s2_ref, b2_ref, wl_ref, bl_ref, o_ref):
  dinv = _dinv_of(degp_ref)
  h2p = jnp.maximum(s2_ref[...] * dinv[:, None] + b2_ref[...], 0.0)
  o_ref[...] = jnp.dot(h2p, wl_ref[...],
                       preferred_element_type=jnp.float32,
                       precision=lax.Precision.HIGHEST) + bl_ref[...]


def _degp_spec():
  return pl.BlockSpec((NC, RB, DW), lambda i: (0, i, 0))


def _mm1(x, W1):
  return pl.pallas_call(
      _mm1_body,
      grid=(GRID,),
      in_specs=[
          pl.BlockSpec((RB, DIM), lambda i: (i, 0)),
          pl.BlockSpec((DIM, DIM), lambda i: (0, 0)),
      ],
      out_specs=pl.BlockSpec((RB, DIM), lambda i: (i, 0)),
      out_shape=jax.ShapeDtypeStruct((N_NODES, DIM), jnp.float32),
  )(x, W1)


def _tc_stage1(degp, h1):
  return pl.pallas_call(
      _tc1_body,
      grid=(GRID,),
      in_specs=[
          _degp_spec(),
          pl.BlockSpec((RB, DIM), lambda i: (i, 0)),
      ],
      out_specs=[pl.BlockSpec((RB, DH), lambda i: (i, 0)),
                 pl.BlockSpec((RB, DH), lambda i: (i, 0))],
      out_shape=[jax.ShapeDtypeStruct((N_NODES, DH), jnp.float32),
                 jax.ShapeDtypeStruct((N_NODES, DH), jnp.float32)],
  )(degp, h1)


def _tc_stage2(degp, s1, b1, W2):
  return pl.pallas_call(
      _tc2_body,
      grid=(GRID,),
      in_specs=[
          _degp_spec(),
          pl.BlockSpec((RB, DIM), lambda i: (i, 0)),
          pl.BlockSpec((1, DIM), lambda i: (0, 0)),
          pl.BlockSpec((DIM, DIM), lambda i: (0, 0)),
      ],
      out_specs=[pl.BlockSpec((RB, DH), lambda i: (i, 0)),
                 pl.BlockSpec((RB, DH), lambda i: (i, 0))],
      out_shape=[jax.ShapeDtypeStruct((N_NODES, DH), jnp.float32),
                 jax.ShapeDtypeStruct((N_NODES, DH), jnp.float32)],
  )(degp, s1, b1, W2)


def _tc_stage3(degp, s2, b2, Wl, bl):
  return pl.pallas_call(
      _tc3_body,
      grid=(GRID,),
      in_specs=[
          _degp_spec(),
          pl.BlockSpec((RB, DIM), lambda i: (i, 0)),
          pl.BlockSpec((1, DIM), lambda i: (0, 0)),
          pl.BlockSpec((DIM, NUM_CLASSES), lambda i: (0, 0)),
          pl.BlockSpec((1, NUM_CLASSES), lambda i: (0, 0)),
      ],
      out_specs=pl.BlockSpec((RB, NUM_CLASSES), lambda i: (i, 0)),
      out_shape=jax.ShapeDtypeStruct((N_NODES, NUM_CLASSES), jnp.float32),
  )(degp, s2, b2, Wl, bl)


def kernel(x, edge_index, W1, b1, W2, b2, Wl, bl):
  ei3 = edge_index.reshape(2, ER, EC)
  h1 = _mm1(x, W1)
  degp = _degrees(edge_index.reshape(2, DER, DEC))
  hs1a, hs1b = _tc_stage1(degp, h1)
  s1 = _propagate(hs1a, hs1b, ei3)
  hs2a, hs2b = _tc_stage2(degp, s1, b1.reshape(1, DIM), W2)
  s2 = _propagate(hs2a, hs2b, ei3)
  return _tc_stage3(degp, s2, b2.reshape(1, DIM), Wl, bl.reshape(1, NUM_CLASSES))


# speedup vs baseline: 1.0240x; 1.0030x over previous
import jax
import jax.numpy as jnp
from jax import lax
from jax.experimental import pallas as pl
from jax.experimental.pallas import tpu as pltpu
from jax.experimental.pallas import tpu_sc as plsc

N_NODES = 10000
N_EDGES = 320000
DIM = 128
NUM_CLASSES = 40

NC = 2
NS = 16
DH = DIM // NC
RPT = N_NODES // NS
EC = 40
ER = N_EDGES // EC
ERT = ER // NS
DEC = 80
DER = N_EDGES // DEC
ERW = DER // (NC * NS)
DW = 16

RB = 2000
GRID = N_NODES // RB


def _sc_mesh():
  return plsc.VectorSubcoreMesh(
      core_axis_name="c", subcore_axis_name="s",
      num_cores=NC, num_subcores=NS)


def _deg_body(ei3, degp, acc, dst2d, ones, zbuf):
  c = lax.axis_index("c")
  s = lax.axis_index("s")
  w = c * NS + s
  r0 = s * RPT

  def fill_ones(i, carry):
    ones[i, :] = jnp.full((DW,), 1.0, dtype=jnp.float32)
    return carry
  lax.fori_loop(0, DEC, fill_ones, 0)

  def fill_zero(i, carry):
    zbuf[i, :] = jnp.zeros((DW,), dtype=jnp.float32)
    return carry
  lax.fori_loop(0, RPT, fill_zero, 0)

  pltpu.sync_copy(zbuf, acc.at[pl.ds(r0, RPT)])
  pltpu.sync_copy(ei3.at[1, pl.ds(w * ERW, ERW)], dst2d)
  plsc.subcore_barrier()

  def chunk(i, carry):
    pltpu.sync_copy(ones, acc.at[dst2d.at[i]], add=True)
    return carry
  lax.fori_loop(0, ERW, chunk, 0)

  plsc.subcore_barrier()
  pltpu.sync_copy(acc.at[pl.ds(r0, RPT)], degp.at[c, pl.ds(r0, RPT)])


def _degrees(ei3):
  f = pl.kernel(
      _deg_body,
      out_type=jax.ShapeDtypeStruct((NC, N_NODES, DW), jnp.float32),
      mesh=_sc_mesh(),
      scratch_types=[
          pltpu.VMEM_SHARED((N_NODES, DW), jnp.float32),
          pltpu.VMEM((ERW, DEC), jnp.int32),
          pltpu.VMEM((DEC, DW), jnp.float32),
          pltpu.VMEM((RPT, DW), jnp.float32),
      ],
      compiler_params=pltpu.CompilerParams(use_tc_tiling_on_sc=False),
      name="gcn_degrees",
  )
  return f(ei3)


def _prop_body(hsa, hsb, ei3, out, table, acc, src2d, dst2d,
               rows0, rows1,
               gsem0, gsem1,
               ssem0, ssem1):
  c = lax.axis_index("c")
  s = lax.axis_index("s")
  r0 = s * RPT
  col0 = c * DH
  rows = (rows0, rows1)
  gsem = (gsem0, gsem1)
  ssem = (ssem0, ssem1)

  def stage(hs):
    pltpu.async_copy(hs.at[pl.ds(r0, RPT)], table.at[pl.ds(r0, RPT)], gsem[0])
    pltpu.async_copy(hs.at[pl.ds(r0, RPT)], acc.at[pl.ds(r0, RPT)], gsem[1])
    pltpu.async_copy(ei3.at[0, pl.ds(s * ERT, ERT)], src2d, ssem[0])
    pltpu.async_copy(ei3.at[1, pl.ds(s * ERT, ERT)], dst2d, ssem[1])
    pltpu.make_async_copy(hs.at[pl.ds(r0, RPT)], table.at[pl.ds(r0, RPT)],
                          gsem[0]).wait()
    pltpu.make_async_copy(hs.at[pl.ds(r0, RPT)], acc.at[pl.ds(r0, RPT)],
                          gsem[1]).wait()
    pltpu.make_async_copy(ei3.at[0, pl.ds(s * ERT, ERT)], src2d,
                          ssem[0]).wait()
    pltpu.make_async_copy(ei3.at[1, pl.ds(s * ERT, ERT)], dst2d,
                          ssem[1]).wait()

  @pl.when(c == 0)
  def _():
    stage(hsa)

  @pl.when(c == 1)
  def _():
    stage(hsb)

  plsc.subcore_barrier()

  def gather_start(i, b):
    pltpu.async_copy(table.at[src2d.at[i]], rows[b], gsem[b])

  def gather_wait(i, b):
    pltpu.make_async_copy(table.at[src2d.at[i]], rows[b], gsem[b]).wait()

  gather_start(0, 0)

  def chunk(it, carry):
    g = it * 2
    gather_wait(g, 0)
    gather_start(g + 1, 1)
    pltpu.sync_copy(rows[0], acc.at[dst2d.at[g]], add=True)
    gather_wait(g + 1, 1)

    @pl.when(g + 2 < ERT)
    def _():
      gather_start(g + 2, 0)

    pltpu.sync_copy(rows[1], acc.at[dst2d.at[g + 1]], add=True)
    return carry
  lax.fori_loop(0, ERT // 2, chunk, 0, unroll=False)

  plsc.subcore_barrier()
  pltpu.sync_copy(acc.at[pl.ds(r0, RPT)],
                  out.at[pl.ds(r0, RPT), pl.ds(col0, DH)])


def _propagate(hsa, hsb, ei3):
  f = pl.kernel(
      _prop_body,
      out_type=jax.ShapeDtypeStruct((N_NODES, DIM), jnp.float32),
      mesh=_sc_mesh(),
      scratch_types=[
          pltpu.VMEM_SHARED((N_NODES, DH), jnp.float32),
          pltpu.VMEM_SHARED((N_NODES, DH), jnp.float32),
          pltpu.VMEM((ERT, EC), jnp.int32),
          pltpu.VMEM((ERT, EC), jnp.int32),
          pltpu.VMEM((EC, DH), jnp.float32),
          pltpu.VMEM((EC, DH), jnp.float32),
          pltpu.SemaphoreType.DMA,
          pltpu.SemaphoreType.DMA,
          pltpu.SemaphoreType.DMA,
          pltpu.SemaphoreType.DMA,
      ],
      compiler_params=pltpu.CompilerParams(use_tc_tiling_on_sc=False),
      name="gcn_propagate",
  )
  return f(hsa, hsb, ei3)


def _dinv_of(degp_ref):
  deg = degp_ref[0, :, 0] + degp_ref[1, :, 0] + 1.0
  return lax.rsqrt(deg)


def _mm1_body(x_ref, w1_ref, o_ref):
  o_ref[...] = jnp.dot(x_ref[...], w1_ref[...],
                       preferred_element_type=jnp.float32,
                       precision=lax.Precision.HIGHEST)


def _tc1_body(degp_ref, h_ref, oa_ref, ob_ref):
  dinv = _dinv_of(degp_ref)
  h = h_ref[...] * dinv[:, None]
  oa_ref[...] = h[:, :DH]
  ob_ref[...] = h[:, DH:]


def _tc2_body(degp_ref, s1_ref, b1_ref, w2_ref, oa_ref, ob_ref):
  dinv = _dinv_of(degp_ref)
  h1p = jnp.maximum(s1_ref[...] * dinv[:, None] + b1_ref[...], 0.0)
  h2 = jnp.dot(h1p, w2_ref[...],
               preferred_element_type=jnp.float32,
               precision=lax.Precision.HIGHEST) * dinv[:, None]
  oa_ref[...] = h2[:, :DH]
  ob_ref[...] = h2[:, DH:]


def _tc3_body(degp_ref, s2_ref, b2_ref, wl_ref, bl_ref, o_ref):
  dinv = _dinv_of(degp_ref)
  h2p = jnp.maximum(s2_ref[...] * dinv[:, None] + b2_ref[...], 0.0)
  o_ref[...] = jnp.dot(h2p, wl_ref[...],
                       preferred_element_type=jnp.float32,
                       precision=lax.Precision.HIGHEST) + bl_ref[...]


def _degp_spec():
  return pl.BlockSpec((NC, RB, DW), lambda i: (0, i, 0))


def _mm1(x, W1):
  return pl.pallas_call(
      _mm1_body,
      grid=(GRID,),
      in_specs=[
          pl.BlockSpec((RB, DIM), lambda i: (i, 0)),
          pl.BlockSpec((DIM, DIM), lambda i: (0, 0)),
      ],
      out_specs=pl.BlockSpec((RB, DIM), lambda i: (i, 0)),
      out_shape=jax.ShapeDtypeStruct((N_NODES, DIM), jnp.float32),
  )(x, W1)


def _tc_stage1(degp, h1):
  return pl.pallas_call(
      _tc1_body,
      grid=(GRID,),
      in_specs=[
          _degp_spec(),
          pl.BlockSpec((RB, DIM), lambda i: (i, 0)),
      ],
      out_specs=[pl.BlockSpec((RB, DH), lambda i: (i, 0)),
                 pl.BlockSpec((RB, DH), lambda i: (i, 0))],
      out_shape=[jax.ShapeDtypeStruct((N_NODES, DH), jnp.float32),
                 jax.ShapeDtypeStruct((N_NODES, DH), jnp.float32)],
  )(degp, h1)


def _tc_stage2(degp, s1, b1, W2):
  return pl.pallas_call(
      _tc2_body,
      grid=(GRID,),
      in_specs=[
          _degp_spec(),
          pl.BlockSpec((RB, DIM), lambda i: (i, 0)),
          pl.BlockSpec((1, DIM), lambda i: (0, 0)),
          pl.BlockSpec((DIM, DIM), lambda i: (0, 0)),
      ],
      out_specs=[pl.BlockSpec((RB, DH), lambda i: (i, 0)),
                 pl.BlockSpec((RB, DH), lambda i: (i, 0))],
      out_shape=[jax.ShapeDtypeStruct((N_NODES, DH), jnp.float32),
                 jax.ShapeDtypeStruct((N_NODES, DH), jnp.float32)],
  )(degp, s1, b1, W2)


def _tc_stage3(degp, s2, b2, Wl, bl):
  return pl.pallas_call(
      _tc3_body,
      grid=(GRID,),
      in_specs=[
          _degp_spec(),
          pl.BlockSpec((RB, DIM), lambda i: (i, 0)),
          pl.BlockSpec((1, DIM), lambda i: (0, 0)),
          pl.BlockSpec((DIM, NUM_CLASSES), lambda i: (0, 0)),
          pl.BlockSpec((1, NUM_CLASSES), lambda i: (0, 0)),
      ],
      out_specs=pl.BlockSpec((RB, NUM_CLASSES), lambda i: (i, 0)),
      out_shape=jax.ShapeDtypeStruct((N_NODES, NUM_CLASSES), jnp.float32),
  )(degp, s2, b2, Wl, bl)


def kernel(x, edge_index, W1, b1, W2, b2, Wl, bl):
  ei3 = edge_index.reshape(2, ER, EC)
  h1 = _mm1(x, W1)
  degp = _degrees(edge_index.reshape(2, DER, DEC))
  hs1a, hs1b = _tc_stage1(degp, h1)
  s1 = _propagate(hs1a, hs1b, ei3)
  hs2a, hs2b = _tc_stage2(degp, s1, b1.reshape(1, DIM), W2)
  s2 = _propagate(hs2a, hs2b, ei3)
  return _tc_stage3(degp, s2, b2.reshape(1, DIM), Wl, bl.reshape(1, NUM_CLASSES))
